# baseline (speedup 1.0000x reference)
import functools

import jax
import jax.numpy as jnp
from jax import lax
from jax.experimental import pallas as pl
from jax.experimental.pallas import tpu as pltpu
from jax.experimental.pallas import tpu_sc as plsc

_N = 50000
_D = 32
_NNZ = 1600000
_TEMP = 0.2
_W1 = 0.2
_L2 = 1e-07
_B = 1024

_NTILE = 16
_SUB = 128
_NSUB = 2
_EROWS = 12800
_RPERT = _EROWS // _NTILE
_NBLK = _RPERT // _NSUB
_NP = 50048
_RPT = _NP // _NTILE
_ZCH = 136


def _spmm_body(rows_hbm, cols_hbm, vals_hbm, eu_hbm, ei_hbm,
               zu_hbm, zi_hbm,
               acc, gidx, sidx, valsb, rowsb, rows16, zbuf,
               isem0, isem1, gsem0, gsem1, ssem0, ssem1):
    cid = lax.axis_index("c")
    sid = lax.axis_index("s")
    isems = (isem0, isem1)
    gsems = (gsem0, gsem1)
    ssems = (ssem0, ssem1)

    def _zb(i, c):
        zbuf[i, pl.ds(0, 16)] = jnp.zeros((16,), jnp.float32)
        zbuf[i, pl.ds(16, 16)] = jnp.zeros((16,), jnp.float32)
        return c
    lax.fori_loop(0, _ZCH, _zb, 0)

    def _zc(k, c):
        pltpu.sync_copy(zbuf, acc.at[pl.ds(sid * _RPT + k * _ZCH, _ZCH)])
        return c
    lax.fori_loop(0, _RPT // _ZCH, _zc, 0)
    plsc.subcore_barrier()

    def _run(gidx_hbm, tab_hbm, sidx_hbm, out_hbm):
        base = sid * _RPERT

        def idx_load(blk, s):
            r0 = base + blk * _NSUB
            pltpu.async_copy(gidx_hbm.at[pl.ds(r0, _NSUB)], gidx.at[s], isems[s])
            pltpu.async_copy(sidx_hbm.at[pl.ds(r0, _NSUB)], sidx.at[s], isems[s])
            pltpu.async_copy(vals_hbm.at[pl.ds(r0, _NSUB)], valsb.at[s], isems[s])

        def idx_wait(s):
            pltpu.make_async_copy(gidx_hbm.at[pl.ds(0, _NSUB)], gidx.at[s], isems[s]).wait()
            pltpu.make_async_copy(sidx_hbm.at[pl.ds(0, _NSUB)], sidx.at[s], isems[s]).wait()
            pltpu.make_async_copy(vals_hbm.at[pl.ds(0, _NSUB)], valsb.at[s], isems[s]).wait()

        def gather_fire(s):
            for j in range(_NSUB):
                pltpu.async_copy(tab_hbm.at[gidx.at[s].at[j]],
                                 rows16.at[s].at[j], gsems[s])

        def gather_wait(s):
            for j in range(_NSUB):
                pltpu.make_async_copy(tab_hbm.at[gidx.at[s].at[j]],
                                      rows16.at[s].at[j], gsems[s]).wait()

        def scatter_fire(s):
            for j in range(_NSUB):
                pltpu.async_copy(rowsb.at[s].at[j], acc.at[sidx.at[s].at[j]],
                                 ssems[s], add=True)

        def scatter_wait(s):
            for j in range(_NSUB):
                pltpu.make_async_copy(rowsb.at[s].at[j], acc.at[sidx.at[s].at[j]],
                                      ssems[s]).wait()

        def scale(s):
            mask = jnp.full((16,), -65536, jnp.int32)
            for j in range(_NSUB):
                def _sc(g, c2, j=j):
                    v16 = valsb[s, j, pl.ds(g * 16, 16)]
                    for u in range(16):
                        e = g * 16 + u
                        y = plsc.bitcast(rows16[s, j, e, :], jnp.int32)
                        lo = plsc.bitcast(lax.shift_left(y, 16), jnp.float32)
                        hi = plsc.bitcast(lax.bitwise_and(y, mask), jnp.float32)
                        vv = lax.broadcast(v16[u], (16,))
                        rowsb[s, j, e, pl.ds(0, 16)] = lo * vv
                        rowsb[s, j, e, pl.ds(16, 16)] = hi * vv
                    return c2
                lax.fori_loop(0, _SUB // 16, _sc, 0)

        idx_load(0, 0)
        idx_wait(0)
        gather_fire(0)
        idx_load(1, 1)

        def pair(p, c):
            b0 = 2 * p
            gather_wait(0)
            scale(0)

            @pl.when(p > 0)
            def _():
                scatter_wait(1)
            idx_wait(1)
            gather_fire(1)
            scatter_fire(0)
            gather_wait(1)
            scale(1)
            scatter_wait(0)

            @pl.when(b0 + 2 < _NBLK)
            def _():
                idx_load(b0 + 2, 0)
                idx_wait(0)
                gather_fire(0)
            scatter_fire(1)

            @pl.when(b0 + 3 < _NBLK)
            def _():
                idx_load(b0 + 3, 1)
            return c

        lax.fori_loop(0, _NBLK // 2, pair, 0)
        scatter_wait(1)
        plsc.subcore_barrier()
        pltpu.sync_copy(acc.at[pl.ds(sid * _RPT, _RPT)],
                        out_hbm.at[pl.ds(sid * _RPT, _RPT)])

    @pl.when(cid == 0)
    def _c0():
        _run(cols_hbm, ei_hbm, rows_hbm, zu_hbm)

    @pl.when(cid == 1)
    def _c1():
        _run(rows_hbm, eu_hbm, cols_hbm, zi_hbm)


def _spmm_call(rows_p, cols_p, vals_p, eu, ei):
    mesh = plsc.VectorSubcoreMesh(core_axis_name="c", subcore_axis_name="s")
    f = pl.kernel(
        _spmm_body,
        out_type=[jax.ShapeDtypeStruct((_NP, _D), jnp.float32)] * 2,
        mesh=mesh,
        scratch_types=[
            pltpu.VMEM_SHARED((_NP, _D), jnp.float32),
            pltpu.VMEM((2, _NSUB, _SUB), jnp.int32),
            pltpu.VMEM((2, _NSUB, _SUB), jnp.int32),
            pltpu.VMEM((2, _NSUB, _SUB), jnp.float32),
            pltpu.VMEM((2, _NSUB, _SUB, _D), jnp.float32),
            pltpu.VMEM((2, _NSUB, _SUB, _D), jnp.bfloat16),
            pltpu.VMEM((_ZCH, _D), jnp.float32),
            pltpu.SemaphoreType.DMA,
            pltpu.SemaphoreType.DMA,
            pltpu.SemaphoreType.DMA,
            pltpu.SemaphoreType.DMA,
            pltpu.SemaphoreType.DMA,
            pltpu.SemaphoreType.DMA,
        ],
        compiler_params=pltpu.CompilerParams(use_tc_tiling_on_sc=False,
                                             needs_layout_passes=False),
    )
    return f(rows_p, cols_p, vals_p, eu, ei)

_CH = 2000
_NCH = _N // _CH


def _loss_body(gu_ref, eu_b_ref, gi_ref, ei_b_ref,
               eu_t_ref, ei_t_ref,
               lossr_ref, negsum_ref, possum_ref,
               accu_ref, acci_ref):
    k = pl.program_id(0)

    @pl.when(k == 0)
    def _init():
        accu_ref[...] = jnp.zeros_like(accu_ref)
        acci_ref[...] = jnp.zeros_like(acci_ref)

    eu_blk = eu_t_ref[0]
    ei_blk = ei_t_ref[0]
    gu = gu_ref[...]
    gi = gi_ref[...]
    inv_t = 1.0 / _TEMP
    su = lax.dot_general(gu, eu_blk, (((1,), (1,)), ((), ())),
                         preferred_element_type=jnp.float32)
    si = lax.dot_general(gi, ei_blk, (((1,), (1,)), ((), ())),
                         preferred_element_type=jnp.float32)
    accu_ref[...] += jnp.sum(jnp.exp(su * inv_t), axis=1, keepdims=True)
    acci_ref[...] += jnp.sum(jnp.exp(si * inv_t), axis=1, keepdims=True)

    @pl.when(k == _NCH - 1)
    def _fin():
        eu_b = eu_b_ref[...]
        ei_b = ei_b_ref[...]
        neg_score = (jnp.mean(jnp.log(accu_ref[...] + 1e-08))
                     + jnp.mean(jnp.log(acci_ref[...] + 1e-08)))
        pos_u = jnp.clip(jnp.sum(gu_ref[...] * eu_b, axis=1) * inv_t, -5.0, 5.0)
        pos_i = jnp.clip(jnp.sum(gi_ref[...] * ei_b, axis=1) * inv_t, -5.0, 5.0)
        pos_score = jnp.mean(pos_u) + jnp.mean(pos_i)
        pos_emb = ei_b[:_B]
        neg_emb = ei_b[_B:]
        d = jnp.sum(eu_b * pos_emb, axis=-1) - jnp.sum(eu_b * neg_emb, axis=-1)
        lossr_ref[...] = jnp.mean(jnp.log(1.0 + jnp.exp(-d))).reshape(1, 1)
        negsum_ref[...] = neg_score.reshape(1, 1)
        possum_ref[...] = pos_score.reshape(1, 1)


def _loss_call(gu_b, eu_b, gi_b, ei_b, eu_t, ei_t):
    full = lambda k: (0, 0)
    out = pl.pallas_call(
        _loss_body,
        grid=(_NCH,),
        in_specs=[
            pl.BlockSpec((_B, _D), full),
            pl.BlockSpec((_B, _D), full),
            pl.BlockSpec((2 * _B, _D), full),
            pl.BlockSpec((2 * _B, _D), full),
            pl.BlockSpec((1, _CH, _D), lambda k: (k, 0, 0)),
            pl.BlockSpec((1, _CH, _D), lambda k: (k, 0, 0)),
        ],
        out_specs=[
            pl.BlockSpec((1, 1), full),
            pl.BlockSpec((1, 1), full),
            pl.BlockSpec((1, 1), full),
        ],
        out_shape=[jax.ShapeDtypeStruct((1, 1), jnp.float32)] * 3,
        scratch_shapes=[
            pltpu.VMEM((_B, 1), jnp.float32),
            pltpu.VMEM((2 * _B, 1), jnp.float32),
        ],
    )(gu_b, eu_b, gi_b, ei_b,
      eu_t.reshape(_NCH, _CH, _D), ei_t.reshape(_NCH, _CH, _D))
    return out


def kernel(users, pos_items, neg_items, E_u_0, E_i_0, adj_indices, adj_values,
           u_mul_s, v_mul_s, ut, vt):
    rows = adj_indices[0]
    cols = adj_indices[1]

    pad = _EROWS * _SUB - _NNZ
    zi = jnp.zeros((pad,), jnp.int32)
    rows_p = jnp.concatenate([rows, zi]).reshape(_EROWS, _SUB)
    cols_p = jnp.concatenate([cols, zi]).reshape(_EROWS, _SUB)
    vals_p = jnp.concatenate([adj_values, jnp.zeros((pad,), jnp.float32)]
                             ).reshape(_EROWS, _SUB)
    tpad = jnp.zeros((_NP - _N, _D), jnp.float32)
    perm = jnp.arange(_D).reshape(2, 16).T.reshape(-1)
    to16 = lambda t: t.astype(jnp.bfloat16)[:, perm]
    eu0_p = jnp.concatenate([E_u_0, tpad])
    ei0_p = jnp.concatenate([E_i_0, tpad])
    Z_u1, Z_i1 = _spmm_call(rows_p, cols_p, vals_p, to16(eu0_p), to16(ei0_p))
    Z_u2, Z_i2 = _spmm_call(rows_p, cols_p, vals_p, to16(Z_u1), to16(Z_i1))
    Z_u1, Z_i1 = Z_u1[:_N], Z_i1[:_N]
    E_u = E_u_0 + Z_u1 + Z_u2[:_N]
    E_i = E_i_0 + Z_i1 + Z_i2[:_N]

    P_u = vt @ (E_i_0 + Z_i1)
    P_i = ut @ (E_u_0 + Z_u1)

    iids = jnp.concatenate([pos_items, neg_items], axis=0)
    gu_b = E_u_0[users] + u_mul_s[users] @ P_u
    gi_b = E_i_0[iids] + v_mul_s[iids] @ P_i
    eu_b = E_u[users]
    ei_b = E_i[iids]

    lossr, negsum, possum = _loss_call(gu_b, eu_b, gi_b, ei_b, E_u, E_i)
    loss_r = lossr[0, 0]
    loss_s = -possum[0, 0] + negsum[0, 0]
    loss_reg = (jnp.sum(E_u_0 ** 2) + jnp.sum(E_i_0 ** 2)) * _L2
    w_loss_s = _W1 * loss_s
    loss = loss_r + w_loss_s + loss_reg
    return (loss, loss_r, w_loss_s)


# speedup vs baseline: 1.5342x; 1.5342x over previous
import functools

import jax
import jax.numpy as jnp
from jax import lax
from jax.experimental import pallas as pl
from jax.experimental.pallas import tpu as pltpu
from jax.experimental.pallas import tpu_sc as plsc

_N = 50000
_D = 32
_NNZ = 1600000
_TEMP = 0.2
_W1 = 0.2
_L2 = 1e-07
_B = 1024

_NTILE = 16
_SUB = 128
_EROWS = 12800
_NBLK = _EROWS // _NTILE
_NP = 50048
_RPT = _NP // _NTILE
_ZCH = 136


def _spmm_body(rows_hbm, cols_hbm, vals_hbm, eu_hbm, ei_hbm,
               zu_hbm, zi_hbm,
               acc, gidx, sidx, valsb, rin, rout, zbuf,
               isem_g, isem_s, isem_v, gsem, ssem):
    cid = lax.axis_index("c")
    sid = lax.axis_index("s")

    def _zb(i, c):
        zbuf[i, pl.ds(0, 16)] = jnp.zeros((16,), jnp.float32)
        zbuf[i, pl.ds(16, 16)] = jnp.zeros((16,), jnp.float32)
        return c
    lax.fori_loop(0, _ZCH, _zb, 0)

    def _zc(k, c):
        pltpu.sync_copy(zbuf, acc.at[pl.ds(sid * _RPT + k * _ZCH, _ZCH)])
        return c
    lax.fori_loop(0, _RPT // _ZCH, _zc, 0)
    plsc.subcore_barrier()

    def _run(gidx_hbm, tab_hbm, sidx_hbm, out_hbm):
        base = sid * _NBLK

        def load_g(blk, il):
            pltpu.async_copy(gidx_hbm.at[pl.ds(base + blk, 1)], gidx.at[il], isem_g)

        def wait_g():
            pltpu.make_async_copy(gidx_hbm.at[pl.ds(0, 1)], gidx.at[0], isem_g).wait()

        def load_s(blk, il):
            pltpu.async_copy(sidx_hbm.at[pl.ds(base + blk, 1)], sidx.at[il], isem_s)

        def wait_s():
            pltpu.make_async_copy(sidx_hbm.at[pl.ds(0, 1)], sidx.at[0], isem_s).wait()

        def load_v(blk, il):
            pltpu.async_copy(vals_hbm.at[pl.ds(base + blk, 1)], valsb.at[il], isem_v)

        def wait_v():
            pltpu.make_async_copy(vals_hbm.at[pl.ds(0, 1)], valsb.at[0], isem_v).wait()

        def gather_fire(ds, il):
            pltpu.async_copy(tab_hbm.at[gidx.at[il].at[0]], rin.at[ds], gsem)

        def gather_wait(ds):
            pltpu.make_async_copy(tab_hbm.at[gidx.at[0].at[0]], rin.at[ds], gsem).wait()

        def scatter_fire(ds, il):
            pltpu.async_copy(rout.at[ds], acc.at[sidx.at[il].at[0]], ssem, add=True)

        def scatter_wait(ds):
            pltpu.make_async_copy(rout.at[ds], acc.at[sidx.at[0].at[0]], ssem).wait()

        def scale(ds, il):
            def _sc(g, c2):
                v16 = valsb[il, 0, pl.ds(g * 16, 16)]
                for u in range(16):
                    e = g * 16 + u
                    vv = lax.broadcast(v16[u], (16,))
                    rout[ds, e, pl.ds(0, 16)] = rin[ds, e, pl.ds(0, 16)] * vv
                    rout[ds, e, pl.ds(16, 16)] = rin[ds, e, pl.ds(16, 16)] * vv
                return c2
            lax.fori_loop(0, _SUB // 16, _sc, 0)

        for k in range(4):
            load_g(k, k)
            load_v(k, k)
        load_s(0, 0)
        load_s(1, 1)
        wait_g()
        gather_fire(0, 0)
        wait_g()
        gather_fire(1, 1)

        def quad(q, c):
            for k in range(4):
                b = 4 * q + k
                ds = k % 2
                gather_wait(ds)

                @pl.when(b >= 2)
                def _():
                    scatter_wait(ds)

                @pl.when(b + 4 < _NBLK)
                def _():
                    load_g(b + 4, k)
                wait_v()
                scale(ds, k)

                @pl.when(b + 4 < _NBLK)
                def _():
                    load_v(b + 4, k)

                @pl.when(b + 2 < _NBLK)
                def _():
                    wait_g()
                    gather_fire(ds, (k + 2) % 4)
                wait_s()
                scatter_fire(ds, k)

                @pl.when(b + 2 < _NBLK)
                def _():
                    load_s(b + 2, (k + 2) % 4)
            return c

        lax.fori_loop(0, _NBLK // 4, quad, 0)
        scatter_wait(0)
        scatter_wait(1)
        plsc.subcore_barrier()
        pltpu.sync_copy(acc.at[pl.ds(sid * _RPT, _RPT)],
                        out_hbm.at[pl.ds(sid * _RPT, _RPT)])

    @pl.when(cid == 0)
    def _c0():
        _run(cols_hbm, ei_hbm, rows_hbm, zu_hbm)

    @pl.when(cid == 1)
    def _c1():
        _run(rows_hbm, eu_hbm, cols_hbm, zi_hbm)


def _spmm_call(rows_p, cols_p, vals_p, eu, ei):
    mesh = plsc.VectorSubcoreMesh(core_axis_name="c", subcore_axis_name="s")
    f = pl.kernel(
        _spmm_body,
        out_type=[jax.ShapeDtypeStruct((_NP, _D), jnp.float32)] * 2,
        mesh=mesh,
        scratch_types=[
            pltpu.VMEM_SHARED((_NP, _D), jnp.float32),
            pltpu.VMEM((4, 1, _SUB), jnp.int32),
            pltpu.VMEM((4, 1, _SUB), jnp.int32),
            pltpu.VMEM((4, 1, _SUB), jnp.float32),
            pltpu.VMEM((2, _SUB, _D), jnp.float32),
            pltpu.VMEM((2, _SUB, _D), jnp.float32),
            pltpu.VMEM((_ZCH, _D), jnp.float32),
            pltpu.SemaphoreType.DMA,
            pltpu.SemaphoreType.DMA,
            pltpu.SemaphoreType.DMA,
            pltpu.SemaphoreType.DMA,
            pltpu.SemaphoreType.DMA,
        ],
        compiler_params=pltpu.CompilerParams(use_tc_tiling_on_sc=False),
    )
    return f(rows_p, cols_p, vals_p, eu, ei)

_CH = 2000
_NCH = _N // _CH


def _loss_body(gu_ref, eu_b_ref, gi_ref, ei_b_ref,
               eu_t_ref, ei_t_ref,
               lossr_ref, negsum_ref, possum_ref,
               accu_ref, acci_ref):
    k = pl.program_id(0)

    @pl.when(k == 0)
    def _init():
        accu_ref[...] = jnp.zeros_like(accu_ref)
        acci_ref[...] = jnp.zeros_like(acci_ref)

    eu_blk = eu_t_ref[0]
    ei_blk = ei_t_ref[0]
    gu = gu_ref[...]
    gi = gi_ref[...]
    inv_t = 1.0 / _TEMP
    su = lax.dot_general(gu, eu_blk, (((1,), (1,)), ((), ())),
                         preferred_element_type=jnp.float32)
    si = lax.dot_general(gi, ei_blk, (((1,), (1,)), ((), ())),
                         preferred_element_type=jnp.float32)
    accu_ref[...] += jnp.sum(jnp.exp(su * inv_t), axis=1, keepdims=True)
    acci_ref[...] += jnp.sum(jnp.exp(si * inv_t), axis=1, keepdims=True)

    @pl.when(k == _NCH - 1)
    def _fin():
        eu_b = eu_b_ref[...]
        ei_b = ei_b_ref[...]
        neg_score = (jnp.mean(jnp.log(accu_ref[...] + 1e-08))
                     + jnp.mean(jnp.log(acci_ref[...] + 1e-08)))
        pos_u = jnp.clip(jnp.sum(gu_ref[...] * eu_b, axis=1) * inv_t, -5.0, 5.0)
        pos_i = jnp.clip(jnp.sum(gi_ref[...] * ei_b, axis=1) * inv_t, -5.0, 5.0)
        pos_score = jnp.mean(pos_u) + jnp.mean(pos_i)
        pos_emb = ei_b[:_B]
        neg_emb = ei_b[_B:]
        d = jnp.sum(eu_b * pos_emb, axis=-1) - jnp.sum(eu_b * neg_emb, axis=-1)
        lossr_ref[...] = jnp.mean(jnp.log(1.0 + jnp.exp(-d))).reshape(1, 1)
        negsum_ref[...] = neg_score.reshape(1, 1)
        possum_ref[...] = pos_score.reshape(1, 1)


def _loss_call(gu_b, eu_b, gi_b, ei_b, eu_t, ei_t):
    full = lambda k: (0, 0)
    out = pl.pallas_call(
        _loss_body,
        grid=(_NCH,),
        in_specs=[
            pl.BlockSpec((_B, _D), full),
            pl.BlockSpec((_B, _D), full),
            pl.BlockSpec((2 * _B, _D), full),
            pl.BlockSpec((2 * _B, _D), full),
            pl.BlockSpec((1, _CH, _D), lambda k: (k, 0, 0)),
            pl.BlockSpec((1, _CH, _D), lambda k: (k, 0, 0)),
        ],
        out_specs=[
            pl.BlockSpec((1, 1), full),
            pl.BlockSpec((1, 1), full),
            pl.BlockSpec((1, 1), full),
        ],
        out_shape=[jax.ShapeDtypeStruct((1, 1), jnp.float32)] * 3,
        scratch_shapes=[
            pltpu.VMEM((_B, 1), jnp.float32),
            pltpu.VMEM((2 * _B, 1), jnp.float32),
        ],
    )(gu_b, eu_b, gi_b, ei_b,
      eu_t.reshape(_NCH, _CH, _D), ei_t.reshape(_NCH, _CH, _D))
    return out


def kernel(users, pos_items, neg_items, E_u_0, E_i_0, adj_indices, adj_values,
           u_mul_s, v_mul_s, ut, vt):
    rows = adj_indices[0]
    cols = adj_indices[1]

    pad = _EROWS * _SUB - _NNZ
    zi = jnp.zeros((pad,), jnp.int32)
    rows_p = jnp.concatenate([rows, zi]).reshape(_EROWS, _SUB)
    cols_p = jnp.concatenate([cols, zi]).reshape(_EROWS, _SUB)
    vals_p = jnp.concatenate([adj_values, jnp.zeros((pad,), jnp.float32)]
                             ).reshape(_EROWS, _SUB)
    tpad = jnp.zeros((_NP - _N, _D), jnp.float32)
    eu0_p = jnp.concatenate([E_u_0, tpad])
    ei0_p = jnp.concatenate([E_i_0, tpad])
    Z_u1, Z_i1 = _spmm_call(rows_p, cols_p, vals_p, eu0_p, ei0_p)
    Z_u2, Z_i2 = _spmm_call(rows_p, cols_p, vals_p, Z_u1, Z_i1)
    Z_u1, Z_i1 = Z_u1[:_N], Z_i1[:_N]
    E_u = E_u_0 + Z_u1 + Z_u2[:_N]
    E_i = E_i_0 + Z_i1 + Z_i2[:_N]

    P_u = vt @ (E_i_0 + Z_i1)
    P_i = ut @ (E_u_0 + Z_u1)

    iids = jnp.concatenate([pos_items, neg_items], axis=0)
    gu_b = E_u_0[users] + u_mul_s[users] @ P_u
    gi_b = E_i_0[iids] + v_mul_s[iids] @ P_i
    eu_b = E_u[users]
    ei_b = E_i[iids]

    lossr, negsum, possum = _loss_call(gu_b, eu_b, gi_b, ei_b, E_u, E_i)
    loss_r = lossr[0, 0]
    loss_s = -possum[0, 0] + negsum[0, 0]
    loss_reg = (jnp.sum(E_u_0 ** 2) + jnp.sum(E_i_0 ** 2)) * _L2
    w_loss_s = _W1 * loss_s
    loss = loss_r + w_loss_s + loss_reg
    return (loss, loss_r, w_loss_s)


# speedup vs baseline: 1.6148x; 1.0526x over previous
import functools

import jax
import jax.numpy as jnp
from jax import lax
from jax.experimental import pallas as pl
from jax.experimental.pallas import tpu as pltpu
from jax.experimental.pallas import tpu_sc as plsc

_N = 50000
_D = 32
_NNZ = 1600000
_TEMP = 0.2
_W1 = 0.2
_L2 = 1e-07
_B = 1024

_NTILE = 16
_SUB = 128
_EROWS = 12800
_NBLK = _EROWS // _NTILE
_NP = 50048
_RPT = _NP // _NTILE
_ZCH = 136


def _spmm_body(rows_hbm, cols_hbm, vals_hbm, eu_hbm, ei_hbm,
               zu_hbm, zi_hbm,
               acc, gidx, sidx, valsb, rin, rout, zbuf,
               isem_g, isem_s, isem_v, gsem, ssem):
    cid = lax.axis_index("c")
    sid = lax.axis_index("s")

    def _zb(i, c):
        zbuf[i, pl.ds(0, 16)] = jnp.zeros((16,), jnp.float32)
        zbuf[i, pl.ds(16, 16)] = jnp.zeros((16,), jnp.float32)
        return c
    lax.fori_loop(0, _ZCH, _zb, 0)

    def _zc(k, c):
        pltpu.sync_copy(zbuf, acc.at[pl.ds(sid * _RPT + k * _ZCH, _ZCH)])
        return c
    lax.fori_loop(0, _RPT // _ZCH, _zc, 0)
    plsc.subcore_barrier()

    def _run(gidx_hbm, tab_hbm, sidx_hbm, out_hbm):
        base = sid * _NBLK

        def load_g(blk, il):
            pltpu.async_copy(gidx_hbm.at[pl.ds(base + blk, 1)], gidx.at[il], isem_g)

        def wait_g():
            pltpu.make_async_copy(gidx_hbm.at[pl.ds(0, 1)], gidx.at[0], isem_g).wait()

        def load_s(blk, il):
            pltpu.async_copy(sidx_hbm.at[pl.ds(base + blk, 1)], sidx.at[il], isem_s)

        def wait_s():
            pltpu.make_async_copy(sidx_hbm.at[pl.ds(0, 1)], sidx.at[0], isem_s).wait()

        def load_v(blk, il):
            pltpu.async_copy(vals_hbm.at[pl.ds(base + blk, 1)], valsb.at[il], isem_v)

        def wait_v():
            pltpu.make_async_copy(vals_hbm.at[pl.ds(0, 1)], valsb.at[0], isem_v).wait()

        def gather_fire(ds, il):
            pltpu.async_copy(tab_hbm.at[gidx.at[il].at[0]], rin.at[ds], gsem)

        def gather_wait(ds):
            pltpu.make_async_copy(tab_hbm.at[gidx.at[0].at[0]], rin.at[ds], gsem).wait()

        def scatter_fire(ds, il):
            pltpu.async_copy(rout.at[ds], acc.at[sidx.at[il].at[0]], ssem, add=True)

        def scatter_wait(ds):
            pltpu.make_async_copy(rout.at[ds], acc.at[sidx.at[0].at[0]], ssem).wait()

        def scale(din, dout, il):
            def _sc(g, c2):
                v16 = valsb[il, 0, pl.ds(g * 16, 16)]
                for u in range(16):
                    e = g * 16 + u
                    vv = lax.broadcast(v16[u], (16,))
                    rout[dout, e, pl.ds(0, 16)] = rin[din, e, pl.ds(0, 16)] * vv
                    rout[dout, e, pl.ds(16, 16)] = rin[din, e, pl.ds(16, 16)] * vv
                return c2
            lax.fori_loop(0, _SUB // 16, _sc, 0)

        for k in range(4):
            load_g(k, k)
            load_v(k, k)
        load_s(0, 0)
        load_s(1, 1)
        for k in range(3):
            wait_g()
            gather_fire(k, k)

        def quad(q, c):
            for k in range(4):
                b = 4 * q + k
                ds = k
                so = k % 2
                gather_wait(ds)

                @pl.when(b >= 2)
                def _():
                    scatter_wait(so)

                @pl.when(b + 4 < _NBLK)
                def _():
                    load_g(b + 4, k)

                @pl.when(b + 3 < _NBLK)
                def _():
                    wait_g()
                    gather_fire((k + 3) % 4, (k + 3) % 4)
                wait_v()
                scale(ds, so, k)

                @pl.when(b + 4 < _NBLK)
                def _():
                    load_v(b + 4, k)
                wait_s()
                scatter_fire(so, k)

                @pl.when(b + 2 < _NBLK)
                def _():
                    load_s(b + 2, (k + 2) % 4)
            return c

        lax.fori_loop(0, _NBLK // 4, quad, 0)
        scatter_wait(0)
        scatter_wait(1)
        plsc.subcore_barrier()
        pltpu.sync_copy(acc.at[pl.ds(sid * _RPT, _RPT)],
                        out_hbm.at[pl.ds(sid * _RPT, _RPT)])

    @pl.when(cid == 0)
    def _c0():
        _run(cols_hbm, ei_hbm, rows_hbm, zu_hbm)

    @pl.when(cid == 1)
    def _c1():
        _run(rows_hbm, eu_hbm, cols_hbm, zi_hbm)


def _spmm_call(rows_p, cols_p, vals_p, eu, ei):
    mesh = plsc.VectorSubcoreMesh(core_axis_name="c", subcore_axis_name="s")
    f = pl.kernel(
        _spmm_body,
        out_type=[jax.ShapeDtypeStruct((_NP, _D), jnp.float32)] * 2,
        mesh=mesh,
        scratch_types=[
            pltpu.VMEM_SHARED((_NP, _D), jnp.float32),
            pltpu.VMEM((4, 1, _SUB), jnp.int32),
            pltpu.VMEM((4, 1, _SUB), jnp.int32),
            pltpu.VMEM((4, 1, _SUB), jnp.float32),
            pltpu.VMEM((4, _SUB, _D), jnp.float32),
            pltpu.VMEM((2, _SUB, _D), jnp.float32),
            pltpu.VMEM((_ZCH, _D), jnp.float32),
            pltpu.SemaphoreType.DMA,
            pltpu.SemaphoreType.DMA,
            pltpu.SemaphoreType.DMA,
            pltpu.SemaphoreType.DMA,
            pltpu.SemaphoreType.DMA,
        ],
        compiler_params=pltpu.CompilerParams(use_tc_tiling_on_sc=False),
    )
    return f(rows_p, cols_p, vals_p, eu, ei)

_CH = 2000
_NCH = _N // _CH


def _loss_body(gu_ref, eu_b_ref, gi_ref, ei_b_ref,
               eu_t_ref, ei_t_ref,
               lossr_ref, negsum_ref, possum_ref,
               accu_ref, acci_ref):
    k = pl.program_id(0)

    @pl.when(k == 0)
    def _init():
        accu_ref[...] = jnp.zeros_like(accu_ref)
        acci_ref[...] = jnp.zeros_like(acci_ref)

    eu_blk = eu_t_ref[0]
    ei_blk = ei_t_ref[0]
    gu = gu_ref[...]
    gi = gi_ref[...]
    inv_t = 1.0 / _TEMP
    su = lax.dot_general(gu, eu_blk, (((1,), (1,)), ((), ())),
                         preferred_element_type=jnp.float32)
    si = lax.dot_general(gi, ei_blk, (((1,), (1,)), ((), ())),
                         preferred_element_type=jnp.float32)
    accu_ref[...] += jnp.sum(jnp.exp(su * inv_t), axis=1, keepdims=True)
    acci_ref[...] += jnp.sum(jnp.exp(si * inv_t), axis=1, keepdims=True)

    @pl.when(k == _NCH - 1)
    def _fin():
        eu_b = eu_b_ref[...]
        ei_b = ei_b_ref[...]
        neg_score = (jnp.mean(jnp.log(accu_ref[...] + 1e-08))
                     + jnp.mean(jnp.log(acci_ref[...] + 1e-08)))
        pos_u = jnp.clip(jnp.sum(gu_ref[...] * eu_b, axis=1) * inv_t, -5.0, 5.0)
        pos_i = jnp.clip(jnp.sum(gi_ref[...] * ei_b, axis=1) * inv_t, -5.0, 5.0)
        pos_score = jnp.mean(pos_u) + jnp.mean(pos_i)
        pos_emb = ei_b[:_B]
        neg_emb = ei_b[_B:]
        d = jnp.sum(eu_b * pos_emb, axis=-1) - jnp.sum(eu_b * neg_emb, axis=-1)
        lossr_ref[...] = jnp.mean(jnp.log(1.0 + jnp.exp(-d))).reshape(1, 1)
        negsum_ref[...] = neg_score.reshape(1, 1)
        possum_ref[...] = pos_score.reshape(1, 1)


def _loss_call(gu_b, eu_b, gi_b, ei_b, eu_t, ei_t):
    full = lambda k: (0, 0)
    out = pl.pallas_call(
        _loss_body,
        grid=(_NCH,),
        in_specs=[
            pl.BlockSpec((_B, _D), full),
            pl.BlockSpec((_B, _D), full),
            pl.BlockSpec((2 * _B, _D), full),
            pl.BlockSpec((2 * _B, _D), full),
            pl.BlockSpec((1, _CH, _D), lambda k: (k, 0, 0)),
            pl.BlockSpec((1, _CH, _D), lambda k: (k, 0, 0)),
        ],
        out_specs=[
            pl.BlockSpec((1, 1), full),
            pl.BlockSpec((1, 1), full),
            pl.BlockSpec((1, 1), full),
        ],
        out_shape=[jax.ShapeDtypeStruct((1, 1), jnp.float32)] * 3,
        scratch_shapes=[
            pltpu.VMEM((_B, 1), jnp.float32),
            pltpu.VMEM((2 * _B, 1), jnp.float32),
        ],
    )(gu_b, eu_b, gi_b, ei_b,
      eu_t.reshape(_NCH, _CH, _D), ei_t.reshape(_NCH, _CH, _D))
    return out


def kernel(users, pos_items, neg_items, E_u_0, E_i_0, adj_indices, adj_values,
           u_mul_s, v_mul_s, ut, vt):
    rows = adj_indices[0]
    cols = adj_indices[1]

    pad = _EROWS * _SUB - _NNZ
    zi = jnp.zeros((pad,), jnp.int32)
    rows_p = jnp.concatenate([rows, zi]).reshape(_EROWS, _SUB)
    cols_p = jnp.concatenate([cols, zi]).reshape(_EROWS, _SUB)
    vals_p = jnp.concatenate([adj_values, jnp.zeros((pad,), jnp.float32)]
                             ).reshape(_EROWS, _SUB)
    tpad = jnp.zeros((_NP - _N, _D), jnp.float32)
    eu0_p = jnp.concatenate([E_u_0, tpad])
    ei0_p = jnp.concatenate([E_i_0, tpad])
    Z_u1, Z_i1 = _spmm_call(rows_p, cols_p, vals_p, eu0_p, ei0_p)
    Z_u2, Z_i2 = _spmm_call(rows_p, cols_p, vals_p, Z_u1, Z_i1)
    Z_u1, Z_i1 = Z_u1[:_N], Z_i1[:_N]
    E_u = E_u_0 + Z_u1 + Z_u2[:_N]
    E_i = E_i_0 + Z_i1 + Z_i2[:_N]

    P_u = vt @ (E_i_0 + Z_i1)
    P_i = ut @ (E_u_0 + Z_u1)

    iids = jnp.concatenate([pos_items, neg_items], axis=0)
    gu_b = E_u_0[users] + u_mul_s[users] @ P_u
    gi_b = E_i_0[iids] + v_mul_s[iids] @ P_i
    eu_b = E_u[users]
    ei_b = E_i[iids]

    lossr, negsum, possum = _loss_call(gu_b, eu_b, gi_b, ei_b, E_u, E_i)
    loss_r = lossr[0, 0]
    loss_s = -possum[0, 0] + negsum[0, 0]
    loss_reg = (jnp.sum(E_u_0 ** 2) + jnp.sum(E_i_0 ** 2)) * _L2
    w_loss_s = _W1 * loss_s
    loss = loss_r + w_loss_s + loss_reg
    return (loss, loss_r, w_loss_s)
